# baseline (speedup 1.0000x reference)
import functools
import jax
import jax.numpy as jnp
from jax import lax
from jax.experimental import pallas as pl
from jax.experimental.pallas import tpu as pltpu
from jax.experimental.pallas import tpu_sc as plsc

NUM_TILES = 50000
NUM_NATIONS = 16
D = 64
NUM_PIECES = 800000

NC, NS, L = 2, 16, 16
P_PAD = 819200
T_PAD = 65536
GROUP = 1024
N_GROUPS = P_PAD // (NC * NS) // GROUP
N_PAIRS = N_GROUPS // 2
STRIPE = T_PAD // NS

TB = 2000
NB = NUM_TILES // TB


def _sc_body(comb_hbm, a2_hbm, s_out, cnt_out, h_out,
             s_sh, cnt_sh, idx0, nat0, val0, idx1, nat1, val1,
             ones_row, a2_v, hist_v, zeros_v, sem0, sem1, sem_sc0, sem_sc1):
    c = lax.axis_index("c")
    s = lax.axis_index("s")
    i16 = lax.iota(jnp.int32, 16)
    ones16 = jnp.ones((16,), jnp.float32)
    z16 = jnp.zeros((16,), jnp.float32)

    for i in range(16):
        hist_v[i, :] = z16
    for i in range(128 // 16):
        ones_row[pl.ds(i * 16, 16)] = ones16

    def _zero(i, _):
        zeros_v[pl.ds(i * 16, 16)] = z16
        return 0
    lax.fori_loop(0, STRIPE // 16, _zero, 0)

    pltpu.sync_copy(a2_hbm, a2_v)

    pltpu.sync_copy(zeros_v, s_sh.at[pl.ds(s * STRIPE, STRIPE)])
    pltpu.sync_copy(zeros_v, cnt_sh.at[pl.ds(s * STRIPE, STRIPE)])
    plsc.subcore_barrier()

    base = (c * NS + s) * (N_GROUPS * GROUP)

    def fire_stage(g, ib, nb, sem):
        off = base + g * GROUP
        pltpu.async_copy(comb_hbm.at[pl.ds(off, GROUP)], ib, sem)
        pltpu.async_copy(comb_hbm.at[pl.ds(P_PAD + off, GROUP)], nb, sem)

    def wait_stage(g, ib, nb, sem):
        off = base + g * GROUP
        pltpu.make_async_copy(comb_hbm.at[pl.ds(off, GROUP)], ib, sem).wait()
        pltpu.make_async_copy(comb_hbm.at[pl.ds(P_PAD + off, GROUP)],
                              nb, sem).wait()

    def compute(nb, vb):
        for k in range(GROUP // 16):
            n16 = nb[pl.ds(k * 16, 16)]
            v16 = plsc.load_gather(a2_v, [n16])
            vb[pl.ds(k * 16, 16)] = v16
            plsc.addupdate_scatter(hist_v, [i16, n16], ones16)

    def fire_scatter(ib, vb, sem):
        for j in range(GROUP // 128):
            pltpu.async_copy(vb.at[pl.ds(j * 128, 128)],
                             s_sh.at[ib.at[pl.ds(j * 128, 128)]],
                             sem, add=True)
            pltpu.async_copy(ones_row,
                             cnt_sh.at[ib.at[pl.ds(j * 128, 128)]],
                             sem, add=True)

    def drain_scatter(ib, vb, sem):
        for j in range(GROUP // 128):
            pltpu.make_async_copy(vb.at[pl.ds(j * 128, 128)],
                                  s_sh.at[ib.at[pl.ds(j * 128, 128)]],
                                  sem).wait()
            pltpu.make_async_copy(ones_row,
                                  cnt_sh.at[ib.at[pl.ds(j * 128, 128)]],
                                  sem).wait()

    fire_stage(0, idx0, nat0, sem0)
    wait_stage(0, idx0, nat0, sem0)
    fire_stage(1, idx1, nat1, sem1)
    compute(nat0, val0)
    fire_scatter(idx0, val0, sem_sc0)

    def _pair(p, _):
        x = 2 * p
        wait_stage(x + 1, idx1, nat1, sem1)
        compute(nat1, val1)
        fire_scatter(idx1, val1, sem_sc1)
        drain_scatter(idx0, val0, sem_sc0)
        fire_stage(x + 2, idx0, nat0, sem0)
        wait_stage(x + 2, idx0, nat0, sem0)
        compute(nat0, val0)
        fire_scatter(idx0, val0, sem_sc0)
        drain_scatter(idx1, val1, sem_sc1)
        fire_stage(x + 3, idx1, nat1, sem1)
        return 0
    lax.fori_loop(0, N_PAIRS - 1, _pair, 0)

    wait_stage(N_GROUPS - 2, idx1, nat1, sem1)
    compute(nat1, val1)
    fire_scatter(idx1, val1, sem_sc1)
    drain_scatter(idx0, val0, sem_sc0)
    fire_stage(N_GROUPS - 1, idx0, nat0, sem0)
    wait_stage(N_GROUPS - 1, idx0, nat0, sem0)
    compute(nat0, val0)
    fire_scatter(idx0, val0, sem_sc0)
    drain_scatter(idx1, val1, sem_sc1)
    drain_scatter(idx0, val0, sem_sc0)

    plsc.subcore_barrier()
    off = c * T_PAD + s * STRIPE
    pltpu.sync_copy(s_sh.at[pl.ds(s * STRIPE, STRIPE)], zeros_v)
    pltpu.sync_copy(zeros_v, s_out.at[pl.ds(off, STRIPE)])
    pltpu.sync_copy(cnt_sh.at[pl.ds(s * STRIPE, STRIPE)], zeros_v)
    pltpu.sync_copy(zeros_v, cnt_out.at[pl.ds(off, STRIPE)])
    pltpu.sync_copy(hist_v, h_out.at[pl.ds((c * NS + s) * 16, 16)])


_sc_hist = functools.partial(
    pl.kernel,
    out_type=[
        jax.ShapeDtypeStruct((NC * T_PAD,), jnp.float32),
        jax.ShapeDtypeStruct((NC * T_PAD,), jnp.float32),
        jax.ShapeDtypeStruct((NC * NS * 16, 16), jnp.float32),
    ],
    mesh=plsc.VectorSubcoreMesh(core_axis_name="c", subcore_axis_name="s"),
    compiler_params=pltpu.CompilerParams(needs_layout_passes=False,
                                         use_tc_tiling_on_sc=True),
    scratch_types=[
        pltpu.VMEM_SHARED((T_PAD,), jnp.float32),
        pltpu.VMEM_SHARED((T_PAD,), jnp.float32),
        pltpu.VMEM((GROUP,), jnp.int32),
        pltpu.VMEM((GROUP,), jnp.int32),
        pltpu.VMEM((GROUP,), jnp.float32),
        pltpu.VMEM((GROUP,), jnp.int32),
        pltpu.VMEM((GROUP,), jnp.int32),
        pltpu.VMEM((GROUP,), jnp.float32),
        pltpu.VMEM((128,), jnp.float32),
        pltpu.VMEM((16,), jnp.float32),
        pltpu.VMEM((16, 16), jnp.float32),
        pltpu.VMEM((STRIPE,), jnp.float32),
        pltpu.SemaphoreType.DMA,
        pltpu.SemaphoreType.DMA,
        pltpu.SemaphoreType.DMA,
        pltpu.SemaphoreType.DMA,
    ],
)(_sc_body)


def _dot(a, b, dims):
    return lax.dot_general(a, b, dimension_numbers=(dims, ((), ())),
                           preferred_element_type=jnp.float32)


def _tc_body(pos_ref, pte_ref, s2_ref, c2_ref, h2_ref,
             tfw_ref, pfw_ref, wrow_ref, pb_ref, tfb_ref, pfb_ref,
             terr_ref, act_ref, nemb_ref, padfix_ref,
             vw1_ref, vb1_ref, vw2_ref, vb2_ref,
             out_ref, accp_ref, accw_ref, accc_ref, val_ref):
    i = pl.program_id(0)
    pos = pos_ref[...]
    pte = pte_ref[...]
    s2 = s2_ref[...]
    c2 = c2_ref[...]
    srow = s2[0, 0:1, :] + s2[0, 1:2, :]
    crow = c2[0, 0:1, :] + c2[0, 1:2, :]
    tfw = tfw_ref[...]
    pfw = pfw_ref[...]
    wrow = wrow_ref[...]

    a1row = _dot(wrow, tfw[:D], ((1,), (1,)))
    a3row = _dot(wrow, pfw[D:], ((1,), (1,)))
    t1 = _dot(a1row, pos, ((1,), (1,)))
    t3 = _dot(a3row, pte, ((1,), (1,)))

    enc_const = (_dot(terr_ref[...][0:1, :], tfw[D:], ((1,), (0,)))
                 + tfb_ref[...] + act_ref[...])
    c0 = jnp.sum(enc_const * wrow) + jnp.sum(pb_ref[...])
    cb = jnp.sum(pfb_ref[...] * wrow)

    out_ref[...] = (t1 + srow + crow * (t3 + cb) + c0)[:, None, :]

    ps = jnp.sum(pos, axis=0, keepdims=True)
    wp = _dot(crow, pte, ((1,), (0,)))
    cs = jnp.sum(crow, axis=1, keepdims=True)

    @pl.when(i == 0)
    def _():
        accp_ref[...] = ps
        accw_ref[...] = wp
        accc_ref[...] = cs

    @pl.when(i > 0)
    def _():
        accp_ref[...] += ps
        accw_ref[...] += wp
        accc_ref[...] += cs

    @pl.when(i == NB - 1)
    def _():
        nh = jnp.sum(h2_ref[...], axis=0, keepdims=True) - padfix_ref[...]
        nhe = _dot(nh, nemb_ref[...], ((1,), (0,)))
        pooled = ((_dot(accp_ref[...], tfw[:D], ((1,), (0,)))
                   + _dot(nhe, pfw[:D], ((1,), (0,)))
                   + _dot(accw_ref[...], pfw[D:], ((1,), (0,)))
                   + jnp.sum(accc_ref[...]) * pfb_ref[...]) * (1.0 / NUM_TILES)
                  + enc_const)
        h = jnp.maximum(_dot(pooled, vw1_ref[...], ((1,), (0,)))
                        + vb1_ref[...], 0.0)
        val_ref[...] = _dot(h, vw2_ref[...], ((1,), (0,))) + vb2_ref[...]


def _tc_call(pos, pte, s2, c2, h2, tfw, pfw, wrow, pb, tfb, pfb,
             terr, act, nemb, padfix, vw1, vb1, vw2, vb2):
    def full(shape):
        nd = len(shape)
        return pl.BlockSpec(shape, lambda i, _nd=nd: (0,) * _nd)

    in_specs = [
        pl.BlockSpec((TB, D), lambda i: (i, 0)),
        pl.BlockSpec((TB, D), lambda i: (i, 0)),
        pl.BlockSpec((1, NC, TB), lambda i: (i, 0, 0)),
        pl.BlockSpec((1, NC, TB), lambda i: (i, 0, 0)),
        full((NC * NS * 16, 16)),
        full((2 * D, D)), full((2 * D, D)), full((1, D)),
        full((1, 1)), full((1, D)), full((1, D)),
        full((2, D)), full((1, D)), full((NUM_NATIONS, D)), full((1, 16)),
        full((D, D)), full((1, D)), full((D, 1)), full((1, 1)),
    ]
    out_specs = [
        pl.BlockSpec((1, 1, TB), lambda i: (i, 0, 0)),
        full((1, D)), full((1, D)), full((1, 1)), full((1, 1)),
    ]
    out_shape = [
        jax.ShapeDtypeStruct((NB, 1, TB), jnp.float32),
        jax.ShapeDtypeStruct((1, D), jnp.float32),
        jax.ShapeDtypeStruct((1, D), jnp.float32),
        jax.ShapeDtypeStruct((1, 1), jnp.float32),
        jax.ShapeDtypeStruct((1, 1), jnp.float32),
    ]
    return pl.pallas_call(
        _tc_body,
        grid=(NB,),
        in_specs=in_specs,
        out_specs=out_specs,
        out_shape=out_shape,
    )(pos, pte, s2, c2, h2, tfw, pfw, wrow, pb, tfb, pfb,
      terr, act, nemb, padfix, vw1, vb1, vw2, vb2)


@jax.jit
def kernel(tile_idxs, terrain_types, nation_idxs, piece_tile_idxs,
           active_nation, tile_pos_emb, terrain_emb, nation_emb,
           piece_tile_emb, tile_fc_W, tile_fc_b, piece_fc_W, piece_fc_b,
           policy_W, policy_b, end_turn_logit, vh_W1, vh_b1, vh_W2, vh_b2):
    pad_n = P_PAD - NUM_PIECES
    pad_idx = NUM_TILES + (jnp.arange(pad_n, dtype=jnp.int32) % 1024)
    comb = jnp.concatenate([piece_tile_idxs, pad_idx,
                            nation_idxs, jnp.zeros((pad_n,), jnp.int32)])

    a2 = (nation_emb @ piece_fc_W[:D]) @ policy_W[:, 0]
    s2, c2, h2 = _sc_hist(comb, a2)

    s2t = s2.reshape(NC, T_PAD)[:, :NUM_TILES].reshape(NC, NB, TB).transpose(1, 0, 2)
    c2t = c2.reshape(NC, T_PAD)[:, :NUM_TILES].reshape(NC, NB, TB).transpose(1, 0, 2)
    padfix = jnp.zeros((1, 16), jnp.float32).at[0, 0].set(float(pad_n))
    act = nation_emb[active_nation][None, :]

    logits, _accp, _accw, _accc, val = _tc_call(
        tile_pos_emb, piece_tile_emb, s2t, c2t, h2,
        tile_fc_W, piece_fc_W, policy_W[:, 0][None, :],
        policy_b.reshape(1, 1), tile_fc_b[None, :], piece_fc_b[None, :],
        terrain_emb, act, nation_emb, padfix,
        vh_W1, vh_b1[None, :], vh_W2, vh_b2.reshape(1, 1))

    policy_logits = jnp.concatenate([logits.reshape(-1),
                                     end_turn_logit[None]])
    return policy_logits, val[0, 0]


# speedup vs baseline: 1.1259x; 1.1259x over previous
import functools
import jax
import jax.numpy as jnp
from jax import lax
from jax.experimental import pallas as pl
from jax.experimental.pallas import tpu as pltpu
from jax.experimental.pallas import tpu_sc as plsc

NUM_TILES = 50000
NUM_NATIONS = 16
D = 64
NUM_PIECES = 800000

NC, NS, L = 2, 16, 16
T_PAD = 50176
GROUP = 1024
N_GROUPS = 25
N_PAIRS = 12
W31_BASE = 31 * N_GROUPS * GROUP
W31_GROUPS = 6
W31_TAIL = NUM_PIECES - W31_BASE - W31_GROUPS * GROUP
STRIPE = T_PAD // NS

TB = 2000
NB = NUM_TILES // TB


def _sc_body(idx_hbm, nat_hbm, a2_hbm, s_out, cnt_out, h_out,
             s_sh, cnt_sh, idx0, nat0, val0, idx1, nat1, val1,
             idx_t, nat_t, val_t,
             ones_row, a2_v, hist_v, zeros_v, sem0, sem1, sem_sc0, sem_sc1):
    c = lax.axis_index("c")
    s = lax.axis_index("s")
    i16 = lax.iota(jnp.int32, 16)
    ones16 = jnp.ones((16,), jnp.float32)
    z16 = jnp.zeros((16,), jnp.float32)

    for i in range(16):
        hist_v[i, :] = z16
    for i in range(128 // 16):
        ones_row[pl.ds(i * 16, 16)] = ones16

    def _zero(i, _):
        zeros_v[pl.ds(i * 16, 16)] = z16
        return 0
    lax.fori_loop(0, STRIPE // 16, _zero, 0)

    pltpu.sync_copy(a2_hbm, a2_v)

    pltpu.sync_copy(zeros_v, s_sh.at[pl.ds(s * STRIPE, STRIPE)])
    pltpu.sync_copy(zeros_v, cnt_sh.at[pl.ds(s * STRIPE, STRIPE)])
    plsc.subcore_barrier()

    wid = c * NS + s
    base = wid * (N_GROUPS * GROUP)

    def fire_stage(g, ib, nb, sem):
        off = base + g * GROUP
        pltpu.async_copy(idx_hbm.at[pl.ds(off, GROUP)], ib, sem)
        pltpu.async_copy(nat_hbm.at[pl.ds(off, GROUP)], nb, sem)

    def wait_stage(g, ib, nb, sem):
        off = base + g * GROUP
        pltpu.make_async_copy(idx_hbm.at[pl.ds(off, GROUP)], ib, sem).wait()
        pltpu.make_async_copy(nat_hbm.at[pl.ds(off, GROUP)], nb, sem).wait()

    def compute(nb, vb):
        for k in range(GROUP // 16):
            n16 = nb[pl.ds(k * 16, 16)]
            v16 = plsc.load_gather(a2_v, [n16])
            vb[pl.ds(k * 16, 16)] = v16
            plsc.addupdate_scatter(hist_v, [i16, n16], ones16)

    def fire_scatter(ib, vb, sem):
        for j in range(GROUP // 128):
            pltpu.async_copy(vb.at[pl.ds(j * 128, 128)],
                             s_sh.at[ib.at[pl.ds(j * 128, 128)]],
                             sem, add=True)
            pltpu.async_copy(ones_row,
                             cnt_sh.at[ib.at[pl.ds(j * 128, 128)]],
                             sem, add=True)

    def drain_scatter(ib, vb, sem):
        for j in range(GROUP // 128):
            pltpu.make_async_copy(vb.at[pl.ds(j * 128, 128)],
                                  s_sh.at[ib.at[pl.ds(j * 128, 128)]],
                                  sem).wait()
            pltpu.make_async_copy(ones_row,
                                  cnt_sh.at[ib.at[pl.ds(j * 128, 128)]],
                                  sem).wait()

    @pl.when(wid < NC * NS - 1)
    def _main():
        fire_stage(0, idx0, nat0, sem0)
        wait_stage(0, idx0, nat0, sem0)
        fire_stage(1, idx1, nat1, sem1)
        compute(nat0, val0)
        fire_scatter(idx0, val0, sem_sc0)

        def _pair(p, _):
            x = 2 * p
            wait_stage(x + 1, idx1, nat1, sem1)
            compute(nat1, val1)
            fire_scatter(idx1, val1, sem_sc1)
            drain_scatter(idx0, val0, sem_sc0)
            fire_stage(x + 2, idx0, nat0, sem0)
            wait_stage(x + 2, idx0, nat0, sem0)
            compute(nat0, val0)
            fire_scatter(idx0, val0, sem_sc0)
            drain_scatter(idx1, val1, sem_sc1)
            fire_stage(x + 3, idx1, nat1, sem1)
            return 0
        lax.fori_loop(0, N_PAIRS - 1, _pair, 0)

        wait_stage(N_GROUPS - 2, idx1, nat1, sem1)
        compute(nat1, val1)
        fire_scatter(idx1, val1, sem_sc1)
        drain_scatter(idx0, val0, sem_sc0)
        fire_stage(N_GROUPS - 1, idx0, nat0, sem0)
        wait_stage(N_GROUPS - 1, idx0, nat0, sem0)
        compute(nat0, val0)
        fire_scatter(idx0, val0, sem_sc0)
        drain_scatter(idx1, val1, sem_sc1)
        drain_scatter(idx0, val0, sem_sc0)

    @pl.when(wid == NC * NS - 1)
    def _last():
        def _g31(g, _):
            off31 = W31_BASE + g * GROUP
            pltpu.sync_copy(idx_hbm.at[pl.ds(off31, GROUP)], idx0)
            pltpu.sync_copy(nat_hbm.at[pl.ds(off31, GROUP)], nat0)
            compute(nat0, val0)
            fire_scatter(idx0, val0, sem_sc0)
            drain_scatter(idx0, val0, sem_sc0)
            return 0
        lax.fori_loop(0, W31_GROUPS, _g31, 0)

        toff = W31_BASE + W31_GROUPS * GROUP
        pltpu.sync_copy(idx_hbm.at[pl.ds(toff, W31_TAIL)], idx_t)
        pltpu.sync_copy(nat_hbm.at[pl.ds(toff, W31_TAIL)], nat_t)
        for k in range(W31_TAIL // 16):
            n16 = nat_t[pl.ds(k * 16, 16)]
            v16 = plsc.load_gather(a2_v, [n16])
            val_t[pl.ds(k * 16, 16)] = v16
            plsc.addupdate_scatter(hist_v, [i16, n16], ones16)
        for j in range(W31_TAIL // 128):
            pltpu.sync_copy(val_t.at[pl.ds(j * 128, 128)],
                            s_sh.at[idx_t.at[pl.ds(j * 128, 128)]], add=True)
            pltpu.sync_copy(ones_row,
                            cnt_sh.at[idx_t.at[pl.ds(j * 128, 128)]], add=True)

    plsc.subcore_barrier()
    off = c * T_PAD + s * STRIPE
    pltpu.sync_copy(s_sh.at[pl.ds(s * STRIPE, STRIPE)], zeros_v)
    pltpu.sync_copy(zeros_v, s_out.at[pl.ds(off, STRIPE)])
    pltpu.sync_copy(cnt_sh.at[pl.ds(s * STRIPE, STRIPE)], zeros_v)
    pltpu.sync_copy(zeros_v, cnt_out.at[pl.ds(off, STRIPE)])
    pltpu.sync_copy(hist_v, h_out.at[pl.ds((c * NS + s) * 16, 16)])


_sc_hist = functools.partial(
    pl.kernel,
    out_type=[
        jax.ShapeDtypeStruct((NC * T_PAD,), jnp.float32),
        jax.ShapeDtypeStruct((NC * T_PAD,), jnp.float32),
        jax.ShapeDtypeStruct((NC * NS * 16, 16), jnp.float32),
    ],
    mesh=plsc.VectorSubcoreMesh(core_axis_name="c", subcore_axis_name="s"),
    compiler_params=pltpu.CompilerParams(needs_layout_passes=False),
    scratch_types=[
        pltpu.VMEM_SHARED((T_PAD,), jnp.float32),
        pltpu.VMEM_SHARED((T_PAD,), jnp.float32),
        pltpu.VMEM((GROUP,), jnp.int32),
        pltpu.VMEM((GROUP,), jnp.int32),
        pltpu.VMEM((GROUP,), jnp.float32),
        pltpu.VMEM((GROUP,), jnp.int32),
        pltpu.VMEM((GROUP,), jnp.int32),
        pltpu.VMEM((GROUP,), jnp.float32),
        pltpu.VMEM((W31_TAIL,), jnp.int32),
        pltpu.VMEM((W31_TAIL,), jnp.int32),
        pltpu.VMEM((W31_TAIL,), jnp.float32),
        pltpu.VMEM((128,), jnp.float32),
        pltpu.VMEM((16,), jnp.float32),
        pltpu.VMEM((16, 16), jnp.float32),
        pltpu.VMEM((STRIPE,), jnp.float32),
        pltpu.SemaphoreType.DMA,
        pltpu.SemaphoreType.DMA,
        pltpu.SemaphoreType.DMA,
        pltpu.SemaphoreType.DMA,
    ],
)(_sc_body)


def _dot(a, b, dims):
    return lax.dot_general(a, b, dimension_numbers=(dims, ((), ())),
                           preferred_element_type=jnp.float32)


def _tc_body(pos_ref, pte_ref, s2_ref, c2_ref, h2_ref,
             tfw_ref, pfw_ref, wrow_ref, pb_ref, tfb_ref, pfb_ref,
             terr_ref, act_ref, nemb_ref, padfix_ref,
             vw1_ref, vb1_ref, vw2_ref, vb2_ref,
             out_ref, accp_ref, accw_ref, accc_ref, val_ref):
    i = pl.program_id(0)
    pos = pos_ref[...]
    pte = pte_ref[...]
    s2 = s2_ref[...]
    c2 = c2_ref[...]
    srow = s2[0, 0:1, :] + s2[0, 1:2, :]
    crow = c2[0, 0:1, :] + c2[0, 1:2, :]
    tfw = tfw_ref[...]
    pfw = pfw_ref[...]
    wrow = wrow_ref[...]

    a1row = _dot(wrow, tfw[:D], ((1,), (1,)))
    a3row = _dot(wrow, pfw[D:], ((1,), (1,)))
    t1 = _dot(a1row, pos, ((1,), (1,)))
    t3 = _dot(a3row, pte, ((1,), (1,)))

    enc_const = (_dot(terr_ref[...][0:1, :], tfw[D:], ((1,), (0,)))
                 + tfb_ref[...] + act_ref[...])
    c0 = jnp.sum(enc_const * wrow) + jnp.sum(pb_ref[...])
    cb = jnp.sum(pfb_ref[...] * wrow)

    out_ref[...] = (t1 + srow + crow * (t3 + cb) + c0)[:, None, :]

    ps = jnp.sum(pos, axis=0, keepdims=True)
    wp = _dot(crow, pte, ((1,), (0,)))
    cs = jnp.sum(crow, axis=1, keepdims=True)

    @pl.when(i == 0)
    def _():
        accp_ref[...] = ps
        accw_ref[...] = wp
        accc_ref[...] = cs

    @pl.when(i > 0)
    def _():
        accp_ref[...] += ps
        accw_ref[...] += wp
        accc_ref[...] += cs

    @pl.when(i == NB - 1)
    def _():
        nh = jnp.sum(h2_ref[...], axis=0, keepdims=True) - padfix_ref[...]
        nhe = _dot(nh, nemb_ref[...], ((1,), (0,)))
        pooled = ((_dot(accp_ref[...], tfw[:D], ((1,), (0,)))
                   + _dot(nhe, pfw[:D], ((1,), (0,)))
                   + _dot(accw_ref[...], pfw[D:], ((1,), (0,)))
                   + jnp.sum(accc_ref[...]) * pfb_ref[...]) * (1.0 / NUM_TILES)
                  + enc_const)
        h = jnp.maximum(_dot(pooled, vw1_ref[...], ((1,), (0,)))
                        + vb1_ref[...], 0.0)
        val_ref[...] = _dot(h, vw2_ref[...], ((1,), (0,))) + vb2_ref[...]


def _tc_call(pos, pte, s2, c2, h2, tfw, pfw, wrow, pb, tfb, pfb,
             terr, act, nemb, padfix, vw1, vb1, vw2, vb2):
    def full(shape):
        nd = len(shape)
        return pl.BlockSpec(shape, lambda i, _nd=nd: (0,) * _nd)

    in_specs = [
        pl.BlockSpec((TB, D), lambda i: (i, 0)),
        pl.BlockSpec((TB, D), lambda i: (i, 0)),
        pl.BlockSpec((1, NC, TB), lambda i: (i, 0, 0)),
        pl.BlockSpec((1, NC, TB), lambda i: (i, 0, 0)),
        full((NC * NS * 16, 16)),
        full((2 * D, D)), full((2 * D, D)), full((1, D)),
        full((1, 1)), full((1, D)), full((1, D)),
        full((2, D)), full((1, D)), full((NUM_NATIONS, D)), full((1, 16)),
        full((D, D)), full((1, D)), full((D, 1)), full((1, 1)),
    ]
    out_specs = [
        pl.BlockSpec((1, 1, TB), lambda i: (i, 0, 0)),
        full((1, D)), full((1, D)), full((1, 1)), full((1, 1)),
    ]
    out_shape = [
        jax.ShapeDtypeStruct((NB, 1, TB), jnp.float32),
        jax.ShapeDtypeStruct((1, D), jnp.float32),
        jax.ShapeDtypeStruct((1, D), jnp.float32),
        jax.ShapeDtypeStruct((1, 1), jnp.float32),
        jax.ShapeDtypeStruct((1, 1), jnp.float32),
    ]
    return pl.pallas_call(
        _tc_body,
        grid=(NB,),
        in_specs=in_specs,
        out_specs=out_specs,
        out_shape=out_shape,
    )(pos, pte, s2, c2, h2, tfw, pfw, wrow, pb, tfb, pfb,
      terr, act, nemb, padfix, vw1, vb1, vw2, vb2)


@jax.jit
def kernel(tile_idxs, terrain_types, nation_idxs, piece_tile_idxs,
           active_nation, tile_pos_emb, terrain_emb, nation_emb,
           piece_tile_emb, tile_fc_W, tile_fc_b, piece_fc_W, piece_fc_b,
           policy_W, policy_b, end_turn_logit, vh_W1, vh_b1, vh_W2, vh_b2):
    a2 = (nation_emb @ piece_fc_W[:D]) @ policy_W[:, 0]
    s2, c2, h2 = _sc_hist(piece_tile_idxs, nation_idxs, a2)

    s2t = s2.reshape(NC, T_PAD)[:, :NUM_TILES].reshape(NC, NB, TB).transpose(1, 0, 2)
    c2t = c2.reshape(NC, T_PAD)[:, :NUM_TILES].reshape(NC, NB, TB).transpose(1, 0, 2)
    padfix = jnp.zeros((1, 16), jnp.float32)
    act = nation_emb[active_nation][None, :]

    logits, _accp, _accw, _accc, val = _tc_call(
        tile_pos_emb, piece_tile_emb, s2t, c2t, h2,
        tile_fc_W, piece_fc_W, policy_W[:, 0][None, :],
        policy_b.reshape(1, 1), tile_fc_b[None, :], piece_fc_b[None, :],
        terrain_emb, act, nation_emb, padfix,
        vh_W1, vh_b1[None, :], vh_W2, vh_b2.reshape(1, 1))

    policy_logits = jnp.concatenate([logits.reshape(-1),
                                     end_turn_logit[None]])
    return policy_logits, val[0, 0]
